# baseline (speedup 1.0000x reference)
import functools

import jax
import jax.numpy as jnp
from jax import lax
from jax.experimental import pallas as pl
from jax.experimental.pallas import tpu as pltpu
from jax.experimental.pallas import tpu_sc as plsc

_B, _L, _H, _E, _K = 2, 4096, 1024, 8, 2
_DFF = 4 * _H
_N = _B * _L
_A = _N * _K
_TM = 512
_S = _A + _E * _TM
_T = _S // _TM
_TF = 1024
_NF = _DFF // _TF
_RB = 512

_NW = 32
_TPW = _N // _NW
_CG = 32
_NCH = _TPW // _CG
_CC = 16
_NCC = _TPW // _CC

_ROUTER_ROWS = 1024


def _router_body(x_ref, wp_ref, logits_ref, e1_ref, e2_ref, w1_ref, w2_ref):
    x = x_ref[...]
    wp = wp_ref[...]
    logits = lax.dot_general(x, wp, (((1,), (1,)), ((), ())),
                             preferred_element_type=jnp.float32)
    logits_ref[...] = logits
    m = jnp.max(logits, axis=1, keepdims=True)
    p = jnp.exp(logits - m)
    iota = lax.broadcasted_iota(jnp.int32, p.shape, 1)
    m1 = jnp.max(p, axis=1, keepdims=True)
    e1 = jnp.min(jnp.where(p == m1, iota, _E), axis=1, keepdims=True)
    p2 = jnp.where(iota == e1, -1.0, p)
    m2 = jnp.max(p2, axis=1, keepdims=True)
    e2 = jnp.min(jnp.where(p2 == m2, iota, _E), axis=1, keepdims=True)
    s = m1 + m2
    e1_ref[...] = e1
    e2_ref[...] = e2
    w1_ref[...] = m1 / s
    w2_ref[...] = m2 / s


def _router(xf, w_prior):
    n_blk = _N // _ROUTER_ROWS
    return pl.pallas_call(
        _router_body,
        grid=(n_blk,),
        in_specs=[
            pl.BlockSpec((_ROUTER_ROWS, _H), lambda i: (i, 0)),
            pl.BlockSpec((_E, _H), lambda i: (0, 0)),
        ],
        out_specs=[
            pl.BlockSpec((_ROUTER_ROWS, _E), lambda i: (i, 0)),
            pl.BlockSpec((_ROUTER_ROWS, 1), lambda i: (i, 0)),
            pl.BlockSpec((_ROUTER_ROWS, 1), lambda i: (i, 0)),
            pl.BlockSpec((_ROUTER_ROWS, 1), lambda i: (i, 0)),
            pl.BlockSpec((_ROUTER_ROWS, 1), lambda i: (i, 0)),
        ],
        out_shape=[
            jax.ShapeDtypeStruct((_N, _E), jnp.float32),
            jax.ShapeDtypeStruct((_N, 1), jnp.int32),
            jax.ShapeDtypeStruct((_N, 1), jnp.int32),
            jax.ShapeDtypeStruct((_N, 1), jnp.float32),
            jax.ShapeDtypeStruct((_N, 1), jnp.float32),
        ],
    )(xf, w_prior)


def _onehot(e):
    return (lax.broadcasted_iota(jnp.int32, (e.shape[0], _E), 1) == e
            ).astype(jnp.float32)


def _scan_body(e1_ref, e2_ref, pos1_ref, pos2_ref, meta_ref):
    cnt = (jnp.sum(_onehot(e1_ref[...]), axis=0, keepdims=True)
           + jnp.sum(_onehot(e2_ref[...]), axis=0, keepdims=True))
    padded = jnp.floor((cnt + (_TM - 1)) / _TM) * _TM
    lt = (lax.broadcasted_iota(jnp.int32, (_E, _E), 0)
          < lax.broadcasted_iota(jnp.int32, (_E, _E), 1)).astype(jnp.float32)
    off = lax.dot_general(padded, lt, (((1,), (0,)), ((), ())),
                          preferred_element_type=jnp.float32)
    total = jnp.sum(padded)

    strict = (lax.broadcasted_iota(jnp.int32, (_RB, _RB), 0)
              > lax.broadcasted_iota(jnp.int32, (_RB, _RB), 1)
              ).astype(jnp.float32)

    def rank_half(e_ref, pos_ref, carry):
        def body(i, carry):
            oh = _onehot(e_ref[pl.ds(i * _RB, _RB), :])
            pre = lax.dot_general(strict, oh, (((1,), (0,)), ((), ())),
                                  preferred_element_type=jnp.float32) + carry
            posf = jnp.sum((pre + off) * oh, axis=1, keepdims=True)
            pos_ref[pl.ds(i * _RB, _RB), :] = posf.astype(jnp.int32)
            return carry + jnp.sum(oh, axis=0, keepdims=True)
        return lax.fori_loop(0, _N // _RB, body, carry)

    carry = rank_half(e1_ref, pos1_ref, jnp.zeros((1, _E), jnp.float32))
    rank_half(e2_ref, pos2_ref, carry)

    tids = lax.broadcasted_iota(jnp.int32, (1, _T), 1).astype(jnp.float32)
    starts = tids * _TM
    used = (starts < total).astype(jnp.float32)
    last_used = total / _TM - 1.0
    tids_eff = jnp.minimum(tids, last_used)
    starts_eff = tids_eff * _TM
    ge = jnp.zeros((1, _T), jnp.float32)
    for e in range(_E):
        ge = ge + (starts_eff >= off[0, e]).astype(jnp.float32)
    texp = ge - 1.0
    meta = jnp.concatenate(
        [texp, used, tids_eff, jnp.zeros((_E - 3, _T), jnp.float32)], axis=0)
    meta_ref[...] = meta.astype(jnp.int32)


def _scan(e1, e2):
    return pl.pallas_call(
        _scan_body,
        out_shape=[
            jax.ShapeDtypeStruct((_N, 1), jnp.int32),
            jax.ShapeDtypeStruct((_N, 1), jnp.int32),
            jax.ShapeDtypeStruct((_E, _T), jnp.int32),
        ],
    )(e1, e2)


@functools.cache
def _sc_kernels():
    mesh = plsc.VectorSubcoreMesh(core_axis_name="c", subcore_axis_name="s")

    @functools.partial(
        pl.kernel,
        out_type=[
            jax.ShapeDtypeStruct((_S, _H), jnp.float32),
            jax.ShapeDtypeStruct((_S,), jnp.float32),
        ],
        mesh=mesh,
        scratch_types=[
            pltpu.VMEM((_NCH, _CG), jnp.int32),
            pltpu.VMEM((_NCH, _CG), jnp.int32),
            pltpu.VMEM((_NCH, _CG), jnp.float32),
            pltpu.VMEM((_NCH, _CG), jnp.float32),
            pltpu.VMEM((2, _CG, _H), jnp.float32),
            pltpu.SemaphoreType.DMA,
            pltpu.SemaphoreType.DMA,
            pltpu.SemaphoreType.DMA,
            pltpu.SemaphoreType.DMA,
        ],
    )
    def dispatch(x_hbm, pos1_hbm, pos2_hbm, w1_hbm, w2_hbm, xs_hbm, ws_hbm,
                 idx1_v, idx2_v, w1b, w2b, xb, sx0, sx1, ss0, ss1):
        wid = lax.axis_index("s") * 2 + lax.axis_index("c")
        base = wid * _TPW
        pltpu.sync_copy(pos1_hbm.at[wid], idx1_v)
        pltpu.sync_copy(pos2_hbm.at[wid], idx2_v)
        pltpu.sync_copy(w1_hbm.at[wid], w1b)
        pltpu.sync_copy(w2_hbm.at[wid], w2b)
        sx = (sx0, sx1)
        ss = (ss0, ss1)

        def load(j, sl):
            return pltpu.async_copy(
                x_hbm.at[pl.ds(base + j * _CG, _CG)], xb.at[sl], sx[sl])

        loads = {0: load(0, 0)}
        scat = {}
        for j in range(_NCH):
            sl = j % 2
            if j >= 1:
                for c in scat.pop(j - 1):
                    c.wait()
            if j + 1 < _NCH:
                loads[j + 1] = load(j + 1, 1 - sl)
            loads.pop(j).wait()
            scat[j] = [
                pltpu.async_copy(xb.at[sl], xs_hbm.at[idx1_v.at[j]], ss[sl]),
                pltpu.async_copy(xb.at[sl], xs_hbm.at[idx2_v.at[j]], ss[sl]),
                pltpu.async_copy(w1b.at[j], ws_hbm.at[idx1_v.at[j]], ss[sl]),
                pltpu.async_copy(w2b.at[j], ws_hbm.at[idx2_v.at[j]], ss[sl]),
            ]
        for c in scat.pop(_NCH - 1):
            c.wait()

    @functools.partial(
        pl.kernel,
        out_type=jax.ShapeDtypeStruct((_N, _H), jnp.float32),
        mesh=mesh,
        scratch_types=[
            pltpu.VMEM((_NCC, _CC), jnp.int32),
            pltpu.VMEM((_NCC, _CC), jnp.int32),
            pltpu.VMEM((2, _CC, _H), jnp.float32),
            pltpu.VMEM((2, _CC, _H), jnp.float32),
            pltpu.VMEM((2, _CC, _H), jnp.float32),
            pltpu.SemaphoreType.DMA,
            pltpu.SemaphoreType.DMA,
            pltpu.SemaphoreType.DMA,
            pltpu.SemaphoreType.DMA,
            pltpu.SemaphoreType.DMA,
            pltpu.SemaphoreType.DMA,
        ],
        compiler_params=pltpu.CompilerParams(needs_layout_passes=False),
    )
    def combine(y_hbm, pos1_hbm, pos2_hbm, out_hbm,
                idx1_v, idx2_v, b1, b2, ob, g1a, g1b, g2a, g2b, oa, obs):
        wid = lax.axis_index("s") * 2 + lax.axis_index("c")
        base = wid * _TPW
        pltpu.sync_copy(pos1_hbm.at[wid], idx1_v)
        pltpu.sync_copy(pos2_hbm.at[wid], idx2_v)
        g1 = (g1a, g1b)
        g2 = (g2a, g2b)
        so = (oa, obs)

        def fire(j, sl):
            return (
                pltpu.async_copy(y_hbm.at[idx1_v.at[j]], b1.at[sl], g1[sl]),
                pltpu.async_copy(y_hbm.at[idx2_v.at[j]], b2.at[sl], g2[sl]),
            )

        pend = {0: fire(0, 0)}
        owr = {}
        for j in range(_NCC):
            sl = j % 2
            if j + 1 < _NCC:
                pend[j + 1] = fire(j + 1, 1 - sl)
            c1, c2 = pend.pop(j)
            c1.wait()
            c2.wait()
            if j >= 2:
                owr.pop(j - 2).wait()

            def rowbody(r, carry):
                for c in range(0, _H, 16):
                    a = b1[sl, r, pl.ds(c, 16)]
                    b = b2[sl, r, pl.ds(c, 16)]
                    ob[sl, r, pl.ds(c, 16)] = a + b
                return carry

            lax.fori_loop(0, _CC, rowbody, 0)
            owr[j] = pltpu.async_copy(
                ob.at[sl], out_hbm.at[pl.ds(base + j * _CC, _CC)], so[sl])
        owr.pop(_NCC - 2).wait()
        owr.pop(_NCC - 1).wait()

    return dispatch, combine


def _expert_body(meta_ref, x_ref, wfc_ref, wproj_ref, ws_ref, out_ref, acc_ref):
    t = pl.program_id(0)
    f = pl.program_id(1)

    @pl.when(meta_ref[1, t] == 1)
    def _():
        h = lax.dot_general(x_ref[...], wfc_ref[0], (((1,), (1,)), ((), ())),
                            preferred_element_type=jnp.float32)
        h = jnp.square(jnp.maximum(h, 0.0))
        p = lax.dot_general(h, wproj_ref[0], (((1,), (1,)), ((), ())),
                            preferred_element_type=jnp.float32)

        @pl.when(f == 0)
        def _():
            acc_ref[...] = p

        @pl.when(f > 0)
        def _():
            acc_ref[...] = acc_ref[...] + p

        @pl.when(f == _NF - 1)
        def _():
            out_ref[...] = acc_ref[...] * ws_ref[...]


def _expert(meta, xs, w_fc, w_proj, ws):
    grid_spec = pltpu.PrefetchScalarGridSpec(
        num_scalar_prefetch=1,
        grid=(_T, _NF),
        in_specs=[
            pl.BlockSpec((_TM, _H), lambda t, f, meta: (meta[2, t], 0)),
            pl.BlockSpec((1, _TF, _H), lambda t, f, meta: (meta[0, t], f, 0)),
            pl.BlockSpec((1, _H, _TF), lambda t, f, meta: (meta[0, t], 0, f)),
            pl.BlockSpec((_TM, 1), lambda t, f, meta: (meta[2, t], 0)),
        ],
        out_specs=pl.BlockSpec((_TM, _H), lambda t, f, meta: (meta[2, t], 0)),
        scratch_shapes=[pltpu.VMEM((_TM, _H), jnp.float32)],
    )
    return pl.pallas_call(
        _expert_body,
        grid_spec=grid_spec,
        out_shape=jax.ShapeDtypeStruct((_S, _H), jnp.float32),
        compiler_params=pltpu.CompilerParams(
            dimension_semantics=("arbitrary", "arbitrary")),
    )(meta, xs, w_fc, w_proj, ws)


def kernel(x, w_prior, w_posterior, w_fc, w_proj):
    del w_posterior
    _dispatch, _combine = _sc_kernels()
    xf = x.reshape(_N, _H)
    logits, e1, e2, w1, w2 = _router(xf, w_prior)
    pos1, pos2, meta = _scan(e1, e2)
    xs, ws = _dispatch(
        xf,
        pos1.reshape(_NW, _NCH, _CG),
        pos2.reshape(_NW, _NCH, _CG),
        w1.reshape(_NW, _NCH, _CG),
        w2.reshape(_NW, _NCH, _CG),
    )
    y = _expert(meta, xs, w_fc, w_proj, ws.reshape(_S, 1))
    out = _combine(
        y,
        pos1.reshape(_NW, _NCC, _CC),
        pos2.reshape(_NW, _NCC, _CC),
    )
    return out.reshape(_B, _L, _H), logits


# speedup vs baseline: 1.0758x; 1.0758x over previous
import functools

import jax
import jax.numpy as jnp
from jax import lax
from jax.experimental import pallas as pl
from jax.experimental.pallas import tpu as pltpu
from jax.experimental.pallas import tpu_sc as plsc

_B, _L, _H, _E, _K = 2, 4096, 1024, 8, 2
_DFF = 4 * _H
_N = _B * _L
_A = _N * _K
_TM = 512
_S = _A + _E * _TM
_T = _S // _TM
_TF = 1024
_NF = _DFF // _TF
_RB = 512

_NW = 32
_TPW = _N // _NW
_CG = 32
_NCH = _TPW // _CG
_CC = 16
_NCC = _TPW // _CC

_ROUTER_ROWS = 1024


def _router_body(x_ref, wp_ref, logits_ref, e1_ref, e2_ref, w1_ref, w2_ref):
    x = x_ref[...]
    wp = wp_ref[...]
    logits = lax.dot_general(x, wp, (((1,), (1,)), ((), ())),
                             preferred_element_type=jnp.float32)
    logits_ref[...] = logits
    m = jnp.max(logits, axis=1, keepdims=True)
    p = jnp.exp(logits - m)
    iota = lax.broadcasted_iota(jnp.int32, p.shape, 1)
    m1 = jnp.max(p, axis=1, keepdims=True)
    e1 = jnp.min(jnp.where(p == m1, iota, _E), axis=1, keepdims=True)
    p2 = jnp.where(iota == e1, -1.0, p)
    m2 = jnp.max(p2, axis=1, keepdims=True)
    e2 = jnp.min(jnp.where(p2 == m2, iota, _E), axis=1, keepdims=True)
    s = m1 + m2
    e1_ref[...] = e1
    e2_ref[...] = e2
    w1_ref[...] = m1 / s
    w2_ref[...] = m2 / s


def _router(xf, w_prior):
    n_blk = _N // _ROUTER_ROWS
    return pl.pallas_call(
        _router_body,
        grid=(n_blk,),
        in_specs=[
            pl.BlockSpec((_ROUTER_ROWS, _H), lambda i: (i, 0)),
            pl.BlockSpec((_E, _H), lambda i: (0, 0)),
        ],
        out_specs=[
            pl.BlockSpec((_ROUTER_ROWS, _E), lambda i: (i, 0)),
            pl.BlockSpec((_ROUTER_ROWS, 1), lambda i: (i, 0)),
            pl.BlockSpec((_ROUTER_ROWS, 1), lambda i: (i, 0)),
            pl.BlockSpec((_ROUTER_ROWS, 1), lambda i: (i, 0)),
            pl.BlockSpec((_ROUTER_ROWS, 1), lambda i: (i, 0)),
        ],
        out_shape=[
            jax.ShapeDtypeStruct((_N, _E), jnp.float32),
            jax.ShapeDtypeStruct((_N, 1), jnp.int32),
            jax.ShapeDtypeStruct((_N, 1), jnp.int32),
            jax.ShapeDtypeStruct((_N, 1), jnp.float32),
            jax.ShapeDtypeStruct((_N, 1), jnp.float32),
        ],
    )(xf, w_prior)


def _onehot(e):
    return (lax.broadcasted_iota(jnp.int32, (e.shape[0], _E), 1) == e
            ).astype(jnp.float32)


def _scan_body(e1_ref, e2_ref, pos1_ref, pos2_ref, meta_ref):
    cnt = (jnp.sum(_onehot(e1_ref[...]), axis=0, keepdims=True)
           + jnp.sum(_onehot(e2_ref[...]), axis=0, keepdims=True))
    padded = jnp.floor((cnt + (_TM - 1)) / _TM) * _TM
    lt = (lax.broadcasted_iota(jnp.int32, (_E, _E), 0)
          < lax.broadcasted_iota(jnp.int32, (_E, _E), 1)).astype(jnp.float32)
    off = lax.dot_general(padded, lt, (((1,), (0,)), ((), ())),
                          preferred_element_type=jnp.float32)
    total = jnp.sum(padded)

    strict = (lax.broadcasted_iota(jnp.int32, (_RB, _RB), 0)
              > lax.broadcasted_iota(jnp.int32, (_RB, _RB), 1)
              ).astype(jnp.float32)

    def rank_half(e_ref, pos_ref, carry):
        def body(i, carry):
            oh = _onehot(e_ref[pl.ds(i * _RB, _RB), :])
            pre = lax.dot_general(strict, oh, (((1,), (0,)), ((), ())),
                                  preferred_element_type=jnp.float32) + carry
            posf = jnp.sum((pre + off) * oh, axis=1, keepdims=True)
            pos_ref[pl.ds(i * _RB, _RB), :] = posf.astype(jnp.int32)
            return carry + jnp.sum(oh, axis=0, keepdims=True)
        return lax.fori_loop(0, _N // _RB, body, carry)

    carry = rank_half(e1_ref, pos1_ref, jnp.zeros((1, _E), jnp.float32))
    rank_half(e2_ref, pos2_ref, carry)

    tids = lax.broadcasted_iota(jnp.int32, (1, _T), 1).astype(jnp.float32)
    starts = tids * _TM
    used = (starts < total).astype(jnp.float32)
    last_used = total / _TM - 1.0
    tids_eff = jnp.minimum(tids, last_used)
    starts_eff = tids_eff * _TM
    ge = jnp.zeros((1, _T), jnp.float32)
    for e in range(_E):
        ge = ge + (starts_eff >= off[0, e]).astype(jnp.float32)
    texp = ge - 1.0
    meta = jnp.concatenate(
        [texp, used, tids_eff, jnp.zeros((_E - 3, _T), jnp.float32)], axis=0)
    meta_ref[...] = meta.astype(jnp.int32)


def _scan(e1, e2):
    return pl.pallas_call(
        _scan_body,
        out_shape=[
            jax.ShapeDtypeStruct((_N, 1), jnp.int32),
            jax.ShapeDtypeStruct((_N, 1), jnp.int32),
            jax.ShapeDtypeStruct((_E, _T), jnp.int32),
        ],
    )(e1, e2)


@functools.cache
def _sc_kernels():
    mesh = plsc.VectorSubcoreMesh(core_axis_name="c", subcore_axis_name="s")

    @functools.partial(
        pl.kernel,
        out_type=jax.ShapeDtypeStruct((_S, _H), jnp.float32),
        mesh=mesh,
        scratch_types=[
            pltpu.VMEM((_NCH, _CG), jnp.int32),
            pltpu.VMEM((_NCH, _CG), jnp.int32),
            pltpu.VMEM((2, _CG, _H), jnp.float32),
            pltpu.SemaphoreType.DMA,
            pltpu.SemaphoreType.DMA,
            pltpu.SemaphoreType.DMA,
            pltpu.SemaphoreType.DMA,
        ],
    )
    def dispatch(x_hbm, pos1_hbm, pos2_hbm, xs_hbm,
                 idx1_v, idx2_v, xb, sx0, sx1, ss0, ss1):
        wid = lax.axis_index("s") * 2 + lax.axis_index("c")
        base = wid * _TPW
        pltpu.sync_copy(pos1_hbm.at[wid], idx1_v)
        pltpu.sync_copy(pos2_hbm.at[wid], idx2_v)
        sx = (sx0, sx1)
        ss = (ss0, ss1)

        def load(j, sl):
            return pltpu.async_copy(
                x_hbm.at[pl.ds(base + j * _CG, _CG)], xb.at[sl], sx[sl])

        loads = {0: load(0, 0)}
        scat = {}
        for j in range(_NCH):
            sl = j % 2
            if j >= 1:
                for c in scat.pop(j - 1):
                    c.wait()
            if j + 1 < _NCH:
                loads[j + 1] = load(j + 1, 1 - sl)
            loads.pop(j).wait()
            scat[j] = [
                pltpu.async_copy(xb.at[sl], xs_hbm.at[idx1_v.at[j]], ss[sl]),
                pltpu.async_copy(xb.at[sl], xs_hbm.at[idx2_v.at[j]], ss[sl]),
            ]
        for c in scat.pop(_NCH - 1):
            c.wait()

    @functools.partial(
        pl.kernel,
        out_type=jax.ShapeDtypeStruct((_N, _H), jnp.float32),
        mesh=mesh,
        scratch_types=[
            pltpu.VMEM((_NCC, _CC), jnp.int32),
            pltpu.VMEM((_NCC, _CC), jnp.int32),
            pltpu.VMEM((_TPW,), jnp.float32),
            pltpu.VMEM((_TPW,), jnp.float32),
            pltpu.VMEM((2, _CC, _H), jnp.float32),
            pltpu.VMEM((2, _CC, _H), jnp.float32),
            pltpu.VMEM((2, _CC, _H), jnp.float32),
            pltpu.SemaphoreType.DMA,
            pltpu.SemaphoreType.DMA,
            pltpu.SemaphoreType.DMA,
            pltpu.SemaphoreType.DMA,
            pltpu.SemaphoreType.DMA,
            pltpu.SemaphoreType.DMA,
        ],
        compiler_params=pltpu.CompilerParams(needs_layout_passes=False),
    )
    def combine(y_hbm, pos1_hbm, pos2_hbm, w1_hbm, w2_hbm, out_hbm,
                idx1_v, idx2_v, w1_v, w2_v, b1, b2, ob, g1a, g1b, g2a, g2b, oa, obs):
        wid = lax.axis_index("s") * 2 + lax.axis_index("c")
        base = wid * _TPW
        pltpu.sync_copy(pos1_hbm.at[wid], idx1_v)
        pltpu.sync_copy(pos2_hbm.at[wid], idx2_v)
        pltpu.sync_copy(w1_hbm.at[pl.ds(base, _TPW)], w1_v)
        pltpu.sync_copy(w2_hbm.at[pl.ds(base, _TPW)], w2_v)
        g1 = (g1a, g1b)
        g2 = (g2a, g2b)
        so = (oa, obs)

        def fire(j, sl):
            return (
                pltpu.async_copy(y_hbm.at[idx1_v.at[j]], b1.at[sl], g1[sl]),
                pltpu.async_copy(y_hbm.at[idx2_v.at[j]], b2.at[sl], g2[sl]),
            )

        pend = {0: fire(0, 0)}
        owr = {}
        for j in range(_NCC):
            sl = j % 2
            if j + 1 < _NCC:
                pend[j + 1] = fire(j + 1, 1 - sl)
            c1, c2 = pend.pop(j)
            c1.wait()
            c2.wait()
            if j >= 2:
                owr.pop(j - 2).wait()

            def rowbody(r, carry):
                tok = j * _CC + r
                wa = plsc.load_gather(w1_v, [jnp.full((16,), tok, jnp.int32)])
                wb = plsc.load_gather(w2_v, [jnp.full((16,), tok, jnp.int32)])
                for c in range(0, _H, 16):
                    a = b1[sl, r, pl.ds(c, 16)]
                    b = b2[sl, r, pl.ds(c, 16)]
                    ob[sl, r, pl.ds(c, 16)] = a * wa + b * wb
                return carry

            lax.fori_loop(0, _CC, rowbody, 0)
            owr[j] = pltpu.async_copy(
                ob.at[sl], out_hbm.at[pl.ds(base + j * _CC, _CC)], so[sl])
        owr.pop(_NCC - 2).wait()
        owr.pop(_NCC - 1).wait()

    return dispatch, combine


def _expert_body(meta_ref, x_ref, wfc_ref, wproj_ref, out_ref, acc_ref):
    t = pl.program_id(0)
    f = pl.program_id(1)

    @pl.when(meta_ref[1, t] == 1)
    def _():
        h = lax.dot_general(x_ref[...], wfc_ref[0], (((1,), (1,)), ((), ())),
                            preferred_element_type=jnp.float32)
        h = jnp.square(jnp.maximum(h, 0.0))
        p = lax.dot_general(h, wproj_ref[0], (((1,), (1,)), ((), ())),
                            preferred_element_type=jnp.float32)

        @pl.when(f == 0)
        def _():
            acc_ref[...] = p

        @pl.when(f > 0)
        def _():
            acc_ref[...] = acc_ref[...] + p

        @pl.when(f == _NF - 1)
        def _():
            out_ref[...] = acc_ref[...]


def _expert(meta, xs, w_fc, w_proj):
    grid_spec = pltpu.PrefetchScalarGridSpec(
        num_scalar_prefetch=1,
        grid=(_T, _NF),
        in_specs=[
            pl.BlockSpec((_TM, _H), lambda t, f, meta: (meta[2, t], 0)),
            pl.BlockSpec((1, _TF, _H), lambda t, f, meta: (meta[0, t], f, 0)),
            pl.BlockSpec((1, _H, _TF), lambda t, f, meta: (meta[0, t], 0, f)),
        ],
        out_specs=pl.BlockSpec((_TM, _H), lambda t, f, meta: (meta[2, t], 0)),
        scratch_shapes=[pltpu.VMEM((_TM, _H), jnp.float32)],
    )
    return pl.pallas_call(
        _expert_body,
        grid_spec=grid_spec,
        out_shape=jax.ShapeDtypeStruct((_S, _H), jnp.float32),
        compiler_params=pltpu.CompilerParams(
            dimension_semantics=("arbitrary", "arbitrary")),
    )(meta, xs, w_fc, w_proj)


def kernel(x, w_prior, w_posterior, w_fc, w_proj):
    del w_posterior
    _dispatch, _combine = _sc_kernels()
    xf = x.reshape(_N, _H)
    logits, e1, e2, w1, w2 = _router(xf, w_prior)
    pos1, pos2, meta = _scan(e1, e2)
    xs = _dispatch(
        xf,
        pos1.reshape(_NW, _NCH, _CG),
        pos2.reshape(_NW, _NCH, _CG),
    )
    y = _expert(meta, xs, w_fc, w_proj)
    out = _combine(
        y,
        pos1.reshape(_NW, _NCC, _CC),
        pos2.reshape(_NW, _NCC, _CC),
        w1.reshape(_N),
        w2.reshape(_N),
    )
    return out.reshape(_B, _L, _H), logits


# speedup vs baseline: 1.1168x; 1.0382x over previous
import functools

import jax
import jax.numpy as jnp
from jax import lax
from jax.experimental import pallas as pl
from jax.experimental.pallas import tpu as pltpu
from jax.experimental.pallas import tpu_sc as plsc

_B, _L, _H, _E, _K = 2, 4096, 1024, 8, 2
_DFF = 4 * _H
_N = _B * _L
_A = _N * _K
_TM = 640
_T = -(-(_A + _E * _TM) // _TM)
_S = _T * _TM
_TF = 1024
_NF = _DFF // _TF
_RB = 512

_NW = 32
_TPW = _N // _NW
_CG = 32
_NCH = _TPW // _CG
_CC = 16
_NCC = _TPW // _CC

_ROUTER_ROWS = 1024


def _router_body(x_ref, wp_ref, logits_ref, e1_ref, e2_ref, w1_ref, w2_ref):
    x = x_ref[...]
    wp = wp_ref[...]
    logits = lax.dot_general(x, wp, (((1,), (1,)), ((), ())),
                             preferred_element_type=jnp.float32)
    logits_ref[...] = logits
    m = jnp.max(logits, axis=1, keepdims=True)
    p = jnp.exp(logits - m)
    iota = lax.broadcasted_iota(jnp.int32, p.shape, 1)
    m1 = jnp.max(p, axis=1, keepdims=True)
    e1 = jnp.min(jnp.where(p == m1, iota, _E), axis=1, keepdims=True)
    p2 = jnp.where(iota == e1, -1.0, p)
    m2 = jnp.max(p2, axis=1, keepdims=True)
    e2 = jnp.min(jnp.where(p2 == m2, iota, _E), axis=1, keepdims=True)
    s = m1 + m2
    e1_ref[...] = e1
    e2_ref[...] = e2
    w1_ref[...] = m1 / s
    w2_ref[...] = m2 / s


def _router(xf, w_prior):
    n_blk = _N // _ROUTER_ROWS
    return pl.pallas_call(
        _router_body,
        grid=(n_blk,),
        in_specs=[
            pl.BlockSpec((_ROUTER_ROWS, _H), lambda i: (i, 0)),
            pl.BlockSpec((_E, _H), lambda i: (0, 0)),
        ],
        out_specs=[
            pl.BlockSpec((_ROUTER_ROWS, _E), lambda i: (i, 0)),
            pl.BlockSpec((_ROUTER_ROWS, 1), lambda i: (i, 0)),
            pl.BlockSpec((_ROUTER_ROWS, 1), lambda i: (i, 0)),
            pl.BlockSpec((_ROUTER_ROWS, 1), lambda i: (i, 0)),
            pl.BlockSpec((_ROUTER_ROWS, 1), lambda i: (i, 0)),
        ],
        out_shape=[
            jax.ShapeDtypeStruct((_N, _E), jnp.float32),
            jax.ShapeDtypeStruct((_N, 1), jnp.int32),
            jax.ShapeDtypeStruct((_N, 1), jnp.int32),
            jax.ShapeDtypeStruct((_N, 1), jnp.float32),
            jax.ShapeDtypeStruct((_N, 1), jnp.float32),
        ],
    )(xf, w_prior)


def _onehot(e):
    return (lax.broadcasted_iota(jnp.int32, (e.shape[0], _E), 1) == e
            ).astype(jnp.float32)


def _scan_body(e1_ref, e2_ref, pos1_ref, pos2_ref, meta_ref):
    cnt = (jnp.sum(_onehot(e1_ref[...]), axis=0, keepdims=True)
           + jnp.sum(_onehot(e2_ref[...]), axis=0, keepdims=True))
    padded = jnp.floor((cnt + (_TM - 1)) / _TM) * _TM
    lt = (lax.broadcasted_iota(jnp.int32, (_E, _E), 0)
          < lax.broadcasted_iota(jnp.int32, (_E, _E), 1)).astype(jnp.float32)
    off = lax.dot_general(padded, lt, (((1,), (0,)), ((), ())),
                          preferred_element_type=jnp.float32)
    total = jnp.sum(padded)

    strict = (lax.broadcasted_iota(jnp.int32, (_RB, _RB), 0)
              > lax.broadcasted_iota(jnp.int32, (_RB, _RB), 1)
              ).astype(jnp.float32)

    def rank_half(e_ref, pos_ref, carry):
        def body(i, carry):
            oh = _onehot(e_ref[pl.ds(i * _RB, _RB), :])
            pre = lax.dot_general(strict, oh, (((1,), (0,)), ((), ())),
                                  preferred_element_type=jnp.float32) + carry
            posf = jnp.sum((pre + off) * oh, axis=1, keepdims=True)
            pos_ref[pl.ds(i * _RB, _RB), :] = posf.astype(jnp.int32)
            return carry + jnp.sum(oh, axis=0, keepdims=True)
        return lax.fori_loop(0, _N // _RB, body, carry)

    carry = rank_half(e1_ref, pos1_ref, jnp.zeros((1, _E), jnp.float32))
    rank_half(e2_ref, pos2_ref, carry)

    tids = lax.broadcasted_iota(jnp.int32, (1, _T), 1).astype(jnp.float32)
    starts = tids * _TM
    used = (starts < total).astype(jnp.float32)
    last_used = total / _TM - 1.0
    tids_eff = jnp.minimum(tids, last_used)
    starts_eff = tids_eff * _TM
    ge = jnp.zeros((1, _T), jnp.float32)
    for e in range(_E):
        ge = ge + (starts_eff >= off[0, e]).astype(jnp.float32)
    texp = ge - 1.0
    meta = jnp.concatenate(
        [texp, used, tids_eff, jnp.zeros((_E - 3, _T), jnp.float32)], axis=0)
    meta_ref[...] = meta.astype(jnp.int32)


def _scan(e1, e2):
    return pl.pallas_call(
        _scan_body,
        out_shape=[
            jax.ShapeDtypeStruct((_N, 1), jnp.int32),
            jax.ShapeDtypeStruct((_N, 1), jnp.int32),
            jax.ShapeDtypeStruct((_E, _T), jnp.int32),
        ],
    )(e1, e2)


@functools.cache
def _sc_kernels():
    mesh = plsc.VectorSubcoreMesh(core_axis_name="c", subcore_axis_name="s")

    @functools.partial(
        pl.kernel,
        out_type=jax.ShapeDtypeStruct((_S, _H), jnp.float32),
        mesh=mesh,
        scratch_types=[
            pltpu.VMEM((_NCH, _CG), jnp.int32),
            pltpu.VMEM((_NCH, _CG), jnp.int32),
            pltpu.VMEM((2, _CG, _H), jnp.float32),
            pltpu.SemaphoreType.DMA,
            pltpu.SemaphoreType.DMA,
            pltpu.SemaphoreType.DMA,
            pltpu.SemaphoreType.DMA,
        ],
    )
    def dispatch(x_hbm, pos1_hbm, pos2_hbm, xs_hbm,
                 idx1_v, idx2_v, xb, sx0, sx1, ss0, ss1):
        wid = lax.axis_index("s") * 2 + lax.axis_index("c")
        base = wid * _TPW
        pltpu.sync_copy(pos1_hbm.at[wid], idx1_v)
        pltpu.sync_copy(pos2_hbm.at[wid], idx2_v)
        sx = (sx0, sx1)
        ss = (ss0, ss1)

        def load(j, sl):
            return pltpu.async_copy(
                x_hbm.at[pl.ds(base + j * _CG, _CG)], xb.at[sl], sx[sl])

        loads = {0: load(0, 0)}
        scat = {}
        for j in range(_NCH):
            sl = j % 2
            if j >= 1:
                for c in scat.pop(j - 1):
                    c.wait()
            if j + 1 < _NCH:
                loads[j + 1] = load(j + 1, 1 - sl)
            loads.pop(j).wait()
            scat[j] = [
                pltpu.async_copy(xb.at[sl], xs_hbm.at[idx1_v.at[j]], ss[sl]),
                pltpu.async_copy(xb.at[sl], xs_hbm.at[idx2_v.at[j]], ss[sl]),
            ]
        for c in scat.pop(_NCH - 1):
            c.wait()

    @functools.partial(
        pl.kernel,
        out_type=jax.ShapeDtypeStruct((_N, _H), jnp.float32),
        mesh=mesh,
        scratch_types=[
            pltpu.VMEM((_NCC, _CC), jnp.int32),
            pltpu.VMEM((_NCC, _CC), jnp.int32),
            pltpu.VMEM((_TPW,), jnp.float32),
            pltpu.VMEM((_TPW,), jnp.float32),
            pltpu.VMEM((2, _CC, _H), jnp.float32),
            pltpu.VMEM((2, _CC, _H), jnp.float32),
            pltpu.VMEM((2, _CC, _H), jnp.float32),
            pltpu.SemaphoreType.DMA,
            pltpu.SemaphoreType.DMA,
            pltpu.SemaphoreType.DMA,
            pltpu.SemaphoreType.DMA,
            pltpu.SemaphoreType.DMA,
            pltpu.SemaphoreType.DMA,
        ],
        compiler_params=pltpu.CompilerParams(needs_layout_passes=False),
    )
    def combine(y_hbm, pos1_hbm, pos2_hbm, w1_hbm, w2_hbm, out_hbm,
                idx1_v, idx2_v, w1_v, w2_v, b1, b2, ob, g1a, g1b, g2a, g2b, oa, obs):
        wid = lax.axis_index("s") * 2 + lax.axis_index("c")
        base = wid * _TPW
        pltpu.sync_copy(pos1_hbm.at[wid], idx1_v)
        pltpu.sync_copy(pos2_hbm.at[wid], idx2_v)
        pltpu.sync_copy(w1_hbm.at[pl.ds(base, _TPW)], w1_v)
        pltpu.sync_copy(w2_hbm.at[pl.ds(base, _TPW)], w2_v)
        g1 = (g1a, g1b)
        g2 = (g2a, g2b)
        so = (oa, obs)

        def fire(j, sl):
            return (
                pltpu.async_copy(y_hbm.at[idx1_v.at[j]], b1.at[sl], g1[sl]),
                pltpu.async_copy(y_hbm.at[idx2_v.at[j]], b2.at[sl], g2[sl]),
            )

        pend = {0: fire(0, 0)}
        owr = {}
        for j in range(_NCC):
            sl = j % 2
            if j + 1 < _NCC:
                pend[j + 1] = fire(j + 1, 1 - sl)
            c1, c2 = pend.pop(j)
            c1.wait()
            c2.wait()
            if j >= 2:
                owr.pop(j - 2).wait()

            def rowbody(r, carry):
                tok = j * _CC + r
                wa = plsc.load_gather(w1_v, [jnp.full((16,), tok, jnp.int32)])
                wb = plsc.load_gather(w2_v, [jnp.full((16,), tok, jnp.int32)])
                for c in range(0, _H, 16):
                    a = b1[sl, r, pl.ds(c, 16)]
                    b = b2[sl, r, pl.ds(c, 16)]
                    ob[sl, r, pl.ds(c, 16)] = a * wa + b * wb
                return carry

            lax.fori_loop(0, _CC, rowbody, 0)
            owr[j] = pltpu.async_copy(
                ob.at[sl], out_hbm.at[pl.ds(base + j * _CC, _CC)], so[sl])
        owr.pop(_NCC - 2).wait()
        owr.pop(_NCC - 1).wait()

    return dispatch, combine


def _expert_body(meta_ref, x_ref, wfc_ref, wproj_ref, out_ref, acc_ref):
    t = pl.program_id(0)
    f = pl.program_id(1)

    @pl.when(meta_ref[1, t] == 1)
    def _():
        h = lax.dot_general(x_ref[...], wfc_ref[0], (((1,), (1,)), ((), ())),
                            preferred_element_type=jnp.float32)
        h = jnp.square(jnp.maximum(h, 0.0))
        p = lax.dot_general(h, wproj_ref[0], (((1,), (1,)), ((), ())),
                            preferred_element_type=jnp.float32)

        @pl.when(f == 0)
        def _():
            acc_ref[...] = p

        @pl.when(f > 0)
        def _():
            acc_ref[...] = acc_ref[...] + p

        @pl.when(f == _NF - 1)
        def _():
            out_ref[...] = acc_ref[...]


def _expert(meta, xs, w_fc, w_proj):
    grid_spec = pltpu.PrefetchScalarGridSpec(
        num_scalar_prefetch=1,
        grid=(_T, _NF),
        in_specs=[
            pl.BlockSpec((_TM, _H), lambda t, f, meta: (meta[2, t], 0)),
            pl.BlockSpec((1, _TF, _H), lambda t, f, meta: (meta[0, t], f, 0)),
            pl.BlockSpec((1, _H, _TF), lambda t, f, meta: (meta[0, t], 0, f)),
        ],
        out_specs=pl.BlockSpec((_TM, _H), lambda t, f, meta: (meta[2, t], 0)),
        scratch_shapes=[pltpu.VMEM((_TM, _H), jnp.float32)],
    )
    return pl.pallas_call(
        _expert_body,
        grid_spec=grid_spec,
        out_shape=jax.ShapeDtypeStruct((_S, _H), jnp.float32),
        compiler_params=pltpu.CompilerParams(
            dimension_semantics=("arbitrary", "arbitrary")),
    )(meta, xs, w_fc, w_proj)


def kernel(x, w_prior, w_posterior, w_fc, w_proj):
    del w_posterior
    _dispatch, _combine = _sc_kernels()
    xf = x.reshape(_N, _H)
    logits, e1, e2, w1, w2 = _router(xf, w_prior)
    pos1, pos2, meta = _scan(e1, e2)
    xs = _dispatch(
        xf,
        pos1.reshape(_NW, _NCH, _CG),
        pos2.reshape(_NW, _NCH, _CG),
    )
    y = _expert(meta, xs, w_fc, w_proj)
    out = _combine(
        y,
        pos1.reshape(_NW, _NCC, _CC),
        pos2.reshape(_NW, _NCC, _CC),
        w1.reshape(_N),
        w2.reshape(_N),
    )
    return out.reshape(_B, _L, _H), logits


# speedup vs baseline: 1.1800x; 1.0566x over previous
import functools

import jax
import jax.numpy as jnp
from jax import lax
from jax.experimental import pallas as pl
from jax.experimental.pallas import tpu as pltpu
from jax.experimental.pallas import tpu_sc as plsc

_B, _L, _H, _E, _K = 2, 4096, 1024, 8, 2
_DFF = 4 * _H
_N = _B * _L
_A = _N * _K
_TM = 768
_T = -(-(_A + _E * _TM) // _TM)
_S = _T * _TM
_TF = 1024
_NF = _DFF // _TF
_RB = 512

_NW = 32
_TPW = _N // _NW
_CG = 32
_NCH = _TPW // _CG
_CC = 16
_NCC = _TPW // _CC

_ROUTER_ROWS = 1024


def _router_body(x_ref, wp_ref, logits_ref, e1_ref, e2_ref, w1_ref, w2_ref):
    x = x_ref[...]
    wp = wp_ref[...]
    logits = lax.dot_general(x, wp, (((1,), (1,)), ((), ())),
                             preferred_element_type=jnp.float32)
    logits_ref[...] = logits
    m = jnp.max(logits, axis=1, keepdims=True)
    p = jnp.exp(logits - m)
    iota = lax.broadcasted_iota(jnp.int32, p.shape, 1)
    m1 = jnp.max(p, axis=1, keepdims=True)
    e1 = jnp.min(jnp.where(p == m1, iota, _E), axis=1, keepdims=True)
    p2 = jnp.where(iota == e1, -1.0, p)
    m2 = jnp.max(p2, axis=1, keepdims=True)
    e2 = jnp.min(jnp.where(p2 == m2, iota, _E), axis=1, keepdims=True)
    s = m1 + m2
    e1_ref[...] = e1
    e2_ref[...] = e2
    w1_ref[...] = m1 / s
    w2_ref[...] = m2 / s


def _router(xf, w_prior):
    n_blk = _N // _ROUTER_ROWS
    return pl.pallas_call(
        _router_body,
        grid=(n_blk,),
        in_specs=[
            pl.BlockSpec((_ROUTER_ROWS, _H), lambda i: (i, 0)),
            pl.BlockSpec((_E, _H), lambda i: (0, 0)),
        ],
        out_specs=[
            pl.BlockSpec((_ROUTER_ROWS, _E), lambda i: (i, 0)),
            pl.BlockSpec((_ROUTER_ROWS, 1), lambda i: (i, 0)),
            pl.BlockSpec((_ROUTER_ROWS, 1), lambda i: (i, 0)),
            pl.BlockSpec((_ROUTER_ROWS, 1), lambda i: (i, 0)),
            pl.BlockSpec((_ROUTER_ROWS, 1), lambda i: (i, 0)),
        ],
        out_shape=[
            jax.ShapeDtypeStruct((_N, _E), jnp.float32),
            jax.ShapeDtypeStruct((_N, 1), jnp.int32),
            jax.ShapeDtypeStruct((_N, 1), jnp.int32),
            jax.ShapeDtypeStruct((_N, 1), jnp.float32),
            jax.ShapeDtypeStruct((_N, 1), jnp.float32),
        ],
    )(xf, w_prior)


def _onehot(e):
    return (lax.broadcasted_iota(jnp.int32, (e.shape[0], _E), 1) == e
            ).astype(jnp.float32)


def _scan_body(e1_ref, e2_ref, pos1_ref, pos2_ref, meta_ref):
    cnt = (jnp.sum(_onehot(e1_ref[...]), axis=0, keepdims=True)
           + jnp.sum(_onehot(e2_ref[...]), axis=0, keepdims=True))
    padded = jnp.floor((cnt + (_TM - 1)) / _TM) * _TM
    lt = (lax.broadcasted_iota(jnp.int32, (_E, _E), 0)
          < lax.broadcasted_iota(jnp.int32, (_E, _E), 1)).astype(jnp.float32)
    off = lax.dot_general(padded, lt, (((1,), (0,)), ((), ())),
                          preferred_element_type=jnp.float32)
    total = jnp.sum(padded)

    strict = (lax.broadcasted_iota(jnp.int32, (_RB, _RB), 0)
              > lax.broadcasted_iota(jnp.int32, (_RB, _RB), 1)
              ).astype(jnp.float32)

    def rank_half(e_ref, pos_ref, carry):
        def body(i, carry):
            oh = _onehot(e_ref[pl.ds(i * _RB, _RB), :])
            pre = lax.dot_general(strict, oh, (((1,), (0,)), ((), ())),
                                  preferred_element_type=jnp.float32) + carry
            posf = jnp.sum((pre + off) * oh, axis=1, keepdims=True)
            pos_ref[pl.ds(i * _RB, _RB), :] = posf.astype(jnp.int32)
            return carry + jnp.sum(oh, axis=0, keepdims=True)
        return lax.fori_loop(0, _N // _RB, body, carry)

    carry = rank_half(e1_ref, pos1_ref, jnp.zeros((1, _E), jnp.float32))
    rank_half(e2_ref, pos2_ref, carry)

    tids = lax.broadcasted_iota(jnp.int32, (1, _T), 1).astype(jnp.float32)
    starts = tids * _TM
    used = (starts < total).astype(jnp.float32)
    last_used = total / _TM - 1.0
    tids_eff = jnp.minimum(tids, last_used)
    starts_eff = tids_eff * _TM
    ge = jnp.zeros((1, _T), jnp.float32)
    for e in range(_E):
        ge = ge + (starts_eff >= off[0, e]).astype(jnp.float32)
    texp = ge - 1.0
    meta = jnp.concatenate(
        [texp, used, tids_eff, jnp.zeros((_E - 3, _T), jnp.float32)], axis=0)
    meta_ref[...] = meta.astype(jnp.int32)


def _scan(e1, e2):
    return pl.pallas_call(
        _scan_body,
        out_shape=[
            jax.ShapeDtypeStruct((_N, 1), jnp.int32),
            jax.ShapeDtypeStruct((_N, 1), jnp.int32),
            jax.ShapeDtypeStruct((_E, _T), jnp.int32),
        ],
    )(e1, e2)


@functools.cache
def _sc_kernels():
    mesh = plsc.VectorSubcoreMesh(core_axis_name="c", subcore_axis_name="s")

    @functools.partial(
        pl.kernel,
        out_type=jax.ShapeDtypeStruct((_S, _H), jnp.float32),
        mesh=mesh,
        scratch_types=[
            pltpu.VMEM((_NCH, _CG), jnp.int32),
            pltpu.VMEM((_NCH, _CG), jnp.int32),
            pltpu.VMEM((2, _CG, _H), jnp.float32),
            pltpu.SemaphoreType.DMA,
            pltpu.SemaphoreType.DMA,
            pltpu.SemaphoreType.DMA,
            pltpu.SemaphoreType.DMA,
        ],
    )
    def dispatch(x_hbm, pos1_hbm, pos2_hbm, xs_hbm,
                 idx1_v, idx2_v, xb, sx0, sx1, ss0, ss1):
        wid = lax.axis_index("s") * 2 + lax.axis_index("c")
        base = wid * _TPW
        pltpu.sync_copy(pos1_hbm.at[wid], idx1_v)
        pltpu.sync_copy(pos2_hbm.at[wid], idx2_v)
        sx = (sx0, sx1)
        ss = (ss0, ss1)

        def load(j, sl):
            return pltpu.async_copy(
                x_hbm.at[pl.ds(base + j * _CG, _CG)], xb.at[sl], sx[sl])

        loads = {0: load(0, 0)}
        scat = {}
        for j in range(_NCH):
            sl = j % 2
            if j >= 1:
                for c in scat.pop(j - 1):
                    c.wait()
            if j + 1 < _NCH:
                loads[j + 1] = load(j + 1, 1 - sl)
            loads.pop(j).wait()
            scat[j] = [
                pltpu.async_copy(xb.at[sl], xs_hbm.at[idx1_v.at[j]], ss[sl]),
                pltpu.async_copy(xb.at[sl], xs_hbm.at[idx2_v.at[j]], ss[sl]),
            ]
        for c in scat.pop(_NCH - 1):
            c.wait()

    @functools.partial(
        pl.kernel,
        out_type=jax.ShapeDtypeStruct((_N, _H), jnp.float32),
        mesh=mesh,
        scratch_types=[
            pltpu.VMEM((_NCC, _CC), jnp.int32),
            pltpu.VMEM((_NCC, _CC), jnp.int32),
            pltpu.VMEM((_TPW,), jnp.float32),
            pltpu.VMEM((_TPW,), jnp.float32),
            pltpu.VMEM((2, _CC, _H), jnp.float32),
            pltpu.VMEM((2, _CC, _H), jnp.float32),
            pltpu.VMEM((2, _CC, _H), jnp.float32),
            pltpu.SemaphoreType.DMA,
            pltpu.SemaphoreType.DMA,
            pltpu.SemaphoreType.DMA,
            pltpu.SemaphoreType.DMA,
            pltpu.SemaphoreType.DMA,
            pltpu.SemaphoreType.DMA,
        ],
        compiler_params=pltpu.CompilerParams(needs_layout_passes=False),
    )
    def combine(y_hbm, pos1_hbm, pos2_hbm, w1_hbm, w2_hbm, out_hbm,
                idx1_v, idx2_v, w1_v, w2_v, b1, b2, ob, g1a, g1b, g2a, g2b, oa, obs):
        wid = lax.axis_index("s") * 2 + lax.axis_index("c")
        base = wid * _TPW
        pltpu.sync_copy(pos1_hbm.at[wid], idx1_v)
        pltpu.sync_copy(pos2_hbm.at[wid], idx2_v)
        pltpu.sync_copy(w1_hbm.at[pl.ds(base, _TPW)], w1_v)
        pltpu.sync_copy(w2_hbm.at[pl.ds(base, _TPW)], w2_v)
        g1 = (g1a, g1b)
        g2 = (g2a, g2b)
        so = (oa, obs)

        def fire(j, sl):
            return (
                pltpu.async_copy(y_hbm.at[idx1_v.at[j]], b1.at[sl], g1[sl]),
                pltpu.async_copy(y_hbm.at[idx2_v.at[j]], b2.at[sl], g2[sl]),
            )

        pend = {0: fire(0, 0)}
        owr = {}
        for j in range(_NCC):
            sl = j % 2
            if j + 1 < _NCC:
                pend[j + 1] = fire(j + 1, 1 - sl)
            c1, c2 = pend.pop(j)
            c1.wait()
            c2.wait()
            if j >= 2:
                owr.pop(j - 2).wait()

            def rowbody(r, carry):
                tok = j * _CC + r
                wa = plsc.load_gather(w1_v, [jnp.full((16,), tok, jnp.int32)])
                wb = plsc.load_gather(w2_v, [jnp.full((16,), tok, jnp.int32)])
                for c in range(0, _H, 16):
                    a = b1[sl, r, pl.ds(c, 16)]
                    b = b2[sl, r, pl.ds(c, 16)]
                    ob[sl, r, pl.ds(c, 16)] = a * wa + b * wb
                return carry

            lax.fori_loop(0, _CC, rowbody, 0)
            owr[j] = pltpu.async_copy(
                ob.at[sl], out_hbm.at[pl.ds(base + j * _CC, _CC)], so[sl])
        owr.pop(_NCC - 2).wait()
        owr.pop(_NCC - 1).wait()

    return dispatch, combine


def _expert_body(meta_ref, x_ref, wfc_ref, wproj_ref, out_ref, acc_ref):
    t = pl.program_id(0)
    f = pl.program_id(1)

    @pl.when(meta_ref[1, t] == 1)
    def _():
        h = lax.dot_general(x_ref[...], wfc_ref[0], (((1,), (1,)), ((), ())),
                            preferred_element_type=jnp.float32)
        h = jnp.square(jnp.maximum(h, 0.0))
        p = lax.dot_general(h, wproj_ref[0], (((1,), (1,)), ((), ())),
                            preferred_element_type=jnp.float32)

        @pl.when(f == 0)
        def _():
            acc_ref[...] = p

        @pl.when(f > 0)
        def _():
            acc_ref[...] = acc_ref[...] + p

        @pl.when(f == _NF - 1)
        def _():
            out_ref[...] = acc_ref[...]


def _expert(meta, xs, w_fc, w_proj):
    grid_spec = pltpu.PrefetchScalarGridSpec(
        num_scalar_prefetch=1,
        grid=(_T, _NF),
        in_specs=[
            pl.BlockSpec((_TM, _H), lambda t, f, meta: (meta[2, t], 0)),
            pl.BlockSpec((1, _TF, _H), lambda t, f, meta: (meta[0, t], f, 0)),
            pl.BlockSpec((1, _H, _TF), lambda t, f, meta: (meta[0, t], 0, f)),
        ],
        out_specs=pl.BlockSpec((_TM, _H), lambda t, f, meta: (meta[2, t], 0)),
        scratch_shapes=[pltpu.VMEM((_TM, _H), jnp.float32)],
    )
    return pl.pallas_call(
        _expert_body,
        grid_spec=grid_spec,
        out_shape=jax.ShapeDtypeStruct((_S, _H), jnp.float32),
        compiler_params=pltpu.CompilerParams(
            dimension_semantics=("arbitrary", "arbitrary")),
    )(meta, xs, w_fc, w_proj)


def kernel(x, w_prior, w_posterior, w_fc, w_proj):
    del w_posterior
    _dispatch, _combine = _sc_kernels()
    xf = x.reshape(_N, _H)
    logits, e1, e2, w1, w2 = _router(xf, w_prior)
    pos1, pos2, meta = _scan(e1, e2)
    xs = _dispatch(
        xf,
        pos1.reshape(_NW, _NCH, _CG),
        pos2.reshape(_NW, _NCH, _CG),
    )
    y = _expert(meta, xs, w_fc, w_proj)
    out = _combine(
        y,
        pos1.reshape(_NW, _NCC, _CC),
        pos2.reshape(_NW, _NCC, _CC),
        w1.reshape(_N),
        w2.reshape(_N),
    )
    return out.reshape(_B, _L, _H), logits
